# baseline (speedup 1.0000x reference)
import functools

import jax
import jax.numpy as jnp
from jax import lax
from jax.experimental import pallas as pl
from jax.experimental.pallas import tpu as pltpu
from jax.experimental.pallas import tpu_sc as plsc

_NUM_CORES = 2
_NUM_SUBCORES = 16
_NUM_WORKERS = _NUM_CORES * _NUM_SUBCORES


@functools.cache
def _make_sc_gather(n_rows, embed_dim):
    rows_per_worker = n_rows // _NUM_WORKERS
    n_chunks = rows_per_worker // 16
    mesh = plsc.VectorSubcoreMesh(core_axis_name="c", subcore_axis_name="s")

    @functools.partial(
        pl.kernel,
        mesh=mesh,
        out_type=jax.ShapeDtypeStruct((n_rows, embed_dim), jnp.float32),
        scratch_types=[
            pltpu.VMEM((rows_per_worker,), jnp.int32),
            pltpu.VMEM((rows_per_worker,), jnp.int32),
            pltpu.VMEM((2, 1, embed_dim), jnp.float32),
            pltpu.SemaphoreType.DMA,
            pltpu.SemaphoreType.DMA,
        ],
    )
    def gather_rows(sidx_hbm, dst_hbm, table_hbm, out_hbm, sidx_v, dst_v, bufs,
                    sem0, sem1):
        wid = lax.axis_index("s") * _NUM_CORES + lax.axis_index("c")
        base = wid * rows_per_worker
        sems = (sem0, sem1)
        pltpu.sync_copy(sidx_hbm.at[pl.ds(base, rows_per_worker)], sidx_v)
        pltpu.sync_copy(dst_hbm.at[pl.ds(base, rows_per_worker)], dst_v)

        def wait_scatter(b):
            pltpu.make_async_copy(
                bufs.at[b], out_hbm.at[pl.ds(0, 1)], sems[b]
            ).wait()

        def chunk_body(c, prev):
            pm2, pm1 = prev
            svec = sidx_v[pl.ds(c * 16, 16)]
            dvec = dst_v[pl.ds(c * 16, 16)]
            for j in range(16):
                b = j % 2
                if j >= 2:
                    wait_scatter(b)
                else:

                    @pl.when(c > 0)
                    def _():
                        wait_scatter(b)

                row = svec[j]
                prev2 = pm2 if j == 0 else (pm1 if j == 1 else svec[j - 2])

                @pl.when(row != prev2)
                def _():
                    pltpu.sync_copy(table_hbm.at[pl.ds(row, 1)], bufs.at[b])

                pltpu.async_copy(
                    bufs.at[b], out_hbm.at[pl.ds(dvec[j], 1)], sems[b]
                )
            return svec[14], svec[15]

        lax.fori_loop(
            0, n_chunks, chunk_body, (jnp.int32(-1), jnp.int32(-1))
        )
        wait_scatter(0)
        wait_scatter(1)

    return gather_rows


def kernel(prefix, table):
    b, t = prefix.shape
    embed_dim = table.shape[1]
    flat_idx = prefix.reshape(-1).astype(jnp.int32)
    sidx, order = lax.sort_key_val(
        flat_idx, jnp.arange(flat_idx.shape[0], dtype=jnp.int32)
    )
    out = _make_sc_gather(b * t, embed_dim)(sidx, order, table)
    return out.reshape(b, t, embed_dim)


# speedup vs baseline: 1.0984x; 1.0984x over previous
import functools

import jax
import jax.numpy as jnp
from jax import lax
from jax.experimental import pallas as pl
from jax.experimental.pallas import tpu as pltpu
from jax.experimental.pallas import tpu_sc as plsc

_NUM_CORES = 2
_NUM_SUBCORES = 16
_NUM_WORKERS = _NUM_CORES * _NUM_SUBCORES


@functools.cache
def _make_sc_gather(n_rows, embed_dim):
    rows_per_worker = n_rows // _NUM_WORKERS
    n_chunks = rows_per_worker // 16
    mesh = plsc.VectorSubcoreMesh(core_axis_name="c", subcore_axis_name="s")

    @functools.partial(
        pl.kernel,
        mesh=mesh,
        out_type=jax.ShapeDtypeStruct((n_rows, embed_dim), jnp.float32),
        scratch_types=[
            pltpu.VMEM((rows_per_worker,), jnp.int32),
            pltpu.VMEM((rows_per_worker,), jnp.int32),
            pltpu.VMEM((1, embed_dim), jnp.float32),
            pltpu.SemaphoreType.DMA,
        ],
    )
    def gather_rows(sidx_hbm, dst_hbm, table_hbm, out_hbm, sidx_v, dst_v, buf,
                    sem):
        wid = lax.axis_index("s") * _NUM_CORES + lax.axis_index("c")
        base = wid * rows_per_worker
        pltpu.sync_copy(sidx_hbm.at[pl.ds(base, rows_per_worker)], sidx_v)
        pltpu.sync_copy(dst_hbm.at[pl.ds(base, rows_per_worker)], dst_v)

        def wait_scatter():
            pltpu.make_async_copy(buf, out_hbm.at[pl.ds(0, 1)], sem).wait()

        def chunk_body(c, prev):
            svec = sidx_v[pl.ds(c * 16, 16)]
            dvec = dst_v[pl.ds(c * 16, 16)]
            for j in range(16):
                if j > 0:
                    wait_scatter()
                else:

                    @pl.when(c > 0)
                    def _():
                        wait_scatter()

                row = svec[j]
                prev_row = prev if j == 0 else svec[j - 1]

                @pl.when(row != prev_row)
                def _():
                    pltpu.sync_copy(table_hbm.at[pl.ds(row, 1)], buf)

                pltpu.async_copy(buf, out_hbm.at[pl.ds(dvec[j], 1)], sem)
            return svec[15]

        lax.fori_loop(0, n_chunks, chunk_body, jnp.int32(-1))
        wait_scatter()

    return gather_rows


def kernel(prefix, table):
    b, t = prefix.shape
    embed_dim = table.shape[1]
    flat_idx = prefix.reshape(-1).astype(jnp.int32)
    sidx, order = lax.sort_key_val(
        flat_idx, jnp.arange(flat_idx.shape[0], dtype=jnp.int32)
    )
    out = _make_sc_gather(b * t, embed_dim)(sidx, order, table)
    return out.reshape(b, t, embed_dim)


# speedup vs baseline: 1.0988x; 1.0004x over previous
import functools

import jax
import jax.numpy as jnp
from jax import lax
from jax.experimental import pallas as pl
from jax.experimental.pallas import tpu as pltpu
from jax.experimental.pallas import tpu_sc as plsc

_NUM_CORES = 2
_NUM_SUBCORES = 16
_NUM_WORKERS = _NUM_CORES * _NUM_SUBCORES


@functools.cache
def _make_sc_gather(n_rows, embed_dim):
    rows_per_worker = n_rows // _NUM_WORKERS
    n_chunks = rows_per_worker // 16
    mesh = plsc.VectorSubcoreMesh(core_axis_name="c", subcore_axis_name="s")

    @functools.partial(
        pl.kernel,
        mesh=mesh,
        out_type=jax.ShapeDtypeStruct((n_rows, embed_dim), jnp.float32),
        scratch_types=[
            pltpu.VMEM((rows_per_worker,), jnp.int32),
            pltpu.VMEM((rows_per_worker,), jnp.int32),
            pltpu.VMEM((1, embed_dim), jnp.float32),
        ],
    )
    def gather_rows(sidx_hbm, dst_hbm, table_hbm, out_hbm, sidx_v, dst_v, buf):
        wid = lax.axis_index("s") * _NUM_CORES + lax.axis_index("c")
        base = wid * rows_per_worker
        pltpu.sync_copy(sidx_hbm.at[pl.ds(base, rows_per_worker)], sidx_v)
        pltpu.sync_copy(dst_hbm.at[pl.ds(base, rows_per_worker)], dst_v)

        def chunk_body(c, prev):
            svec = sidx_v[pl.ds(c * 16, 16)]
            dvec = dst_v[pl.ds(c * 16, 16)]
            for j in range(16):
                row = svec[j]
                prev_row = prev if j == 0 else svec[j - 1]

                @pl.when(row != prev_row)
                def _():
                    pltpu.sync_copy(table_hbm.at[pl.ds(row, 1)], buf)

                pltpu.sync_copy(buf, out_hbm.at[pl.ds(dvec[j], 1)])
            return svec[15]

        lax.fori_loop(0, n_chunks, chunk_body, jnp.int32(-1))

    return gather_rows


def kernel(prefix, table):
    b, t = prefix.shape
    embed_dim = table.shape[1]
    flat_idx = prefix.reshape(-1).astype(jnp.int32)
    sidx, order = lax.sort_key_val(
        flat_idx, jnp.arange(flat_idx.shape[0], dtype=jnp.int32)
    )
    out = _make_sc_gather(b * t, embed_dim)(sidx, order, table)
    return out.reshape(b, t, embed_dim)


# speedup vs baseline: 1.1014x; 1.0024x over previous
import functools

import jax
import jax.numpy as jnp
from jax import lax
from jax.experimental import pallas as pl
from jax.experimental.pallas import tpu as pltpu
from jax.experimental.pallas import tpu_sc as plsc

_NUM_CORES = 2
_NUM_SUBCORES = 16
_NUM_WORKERS = _NUM_CORES * _NUM_SUBCORES


@functools.cache
def _make_sc_gather(n_rows, embed_dim):
    rows_per_worker = n_rows // _NUM_WORKERS
    n_chunks = rows_per_worker // 16
    mesh = plsc.VectorSubcoreMesh(core_axis_name="c", subcore_axis_name="s")

    @functools.partial(
        pl.kernel,
        mesh=mesh,
        out_type=jax.ShapeDtypeStruct((n_rows, embed_dim), jnp.float32),
        scratch_types=[
            pltpu.VMEM((2 * rows_per_worker,), jnp.int32),
            pltpu.VMEM((1, embed_dim), jnp.float32),
        ],
    )
    def gather_rows(pairs_hbm, table_hbm, out_hbm, pairs_v, buf):
        wid = lax.axis_index("s") * _NUM_CORES + lax.axis_index("c")
        pltpu.sync_copy(
            pairs_hbm.at[pl.ds(wid * 2 * rows_per_worker, 2 * rows_per_worker)],
            pairs_v,
        )

        def chunk_body(c, prev):
            svec = pairs_v[pl.ds(c * 16, 16)]
            dvec = pairs_v[pl.ds(rows_per_worker + c * 16, 16)]
            for j in range(16):
                row = svec[j]
                prev_row = prev if j == 0 else svec[j - 1]

                @pl.when(row != prev_row)
                def _():
                    pltpu.sync_copy(table_hbm.at[pl.ds(row, 1)], buf)

                pltpu.sync_copy(buf, out_hbm.at[pl.ds(dvec[j], 1)])
            return svec[15]

        lax.fori_loop(0, n_chunks, chunk_body, jnp.int32(-1))

    return gather_rows


def kernel(prefix, table):
    b, t = prefix.shape
    embed_dim = table.shape[1]
    flat_idx = prefix.reshape(-1).astype(jnp.int32)
    sidx, order = lax.sort_key_val(
        flat_idx, jnp.arange(flat_idx.shape[0], dtype=jnp.int32)
    )
    rpw = (b * t) // _NUM_WORKERS
    pairs = jnp.concatenate(
        [sidx.reshape(_NUM_WORKERS, rpw), order.reshape(_NUM_WORKERS, rpw)],
        axis=1,
    ).reshape(-1)
    out = _make_sc_gather(b * t, embed_dim)(pairs, table)
    return out.reshape(b, t, embed_dim)
